# baseline (speedup 1.0000x reference)
import functools

import jax
import jax.numpy as jnp
from jax.experimental import pallas as pl
from jax.experimental.pallas import tpu as pltpu

BATCH = 4
SEQ_LEN = 8192
EMBED_DIM = 1024

SEQ_TILE = 512
SEQ_TILES = SEQ_LEN // SEQ_TILE


def _add_body(x_ref, t_ref, o_ref):
    o_ref[...] = x_ref[...] + t_ref[...][None]


@jax.jit
def _tc_add(x, pos_table):
    return pl.pallas_call(
        _add_body,
        grid=(SEQ_TILES,),
        in_specs=[
            pl.BlockSpec((BATCH, SEQ_TILE, EMBED_DIM), lambda s: (0, s, 0)),
            pl.BlockSpec((SEQ_TILE, EMBED_DIM), lambda s: (s, 0)),
        ],
        out_specs=pl.BlockSpec((BATCH, SEQ_TILE, EMBED_DIM), lambda s: (0, s, 0)),
        out_shape=jax.ShapeDtypeStruct((BATCH, SEQ_LEN, EMBED_DIM), jnp.float32),
        compiler_params=pltpu.CompilerParams(
            dimension_semantics=("arbitrary",),
        ),
    )(x, pos_table)


def kernel(x, pos_table):
    return _tc_add(x, pos_table)


# speedup vs baseline: 1.0029x; 1.0029x over previous
import functools

import jax
import jax.numpy as jnp
from jax.experimental import pallas as pl
from jax.experimental.pallas import tpu as pltpu

BATCH = 4
SEQ_LEN = 8192
EMBED_DIM = 1024

SEQ_TILE = 2048
SEQ_TILES = SEQ_LEN // SEQ_TILE


def _add_body(x_ref, t_ref, o_ref):
    o_ref[...] = x_ref[...] + t_ref[...][None]


@jax.jit
def _tc_add(x, pos_table):
    return pl.pallas_call(
        _add_body,
        grid=(SEQ_TILES, BATCH),
        in_specs=[
            pl.BlockSpec((1, SEQ_TILE, EMBED_DIM), lambda s, b: (b, s, 0)),
            pl.BlockSpec((SEQ_TILE, EMBED_DIM), lambda s, b: (s, 0)),
        ],
        out_specs=pl.BlockSpec((1, SEQ_TILE, EMBED_DIM), lambda s, b: (b, s, 0)),
        out_shape=jax.ShapeDtypeStruct((BATCH, SEQ_LEN, EMBED_DIM), jnp.float32),
        compiler_params=pltpu.CompilerParams(
            dimension_semantics=("arbitrary", "arbitrary"),
        ),
    )(x, pos_table)


def kernel(x, pos_table):
    return _tc_add(x, pos_table)


# speedup vs baseline: 1.0032x; 1.0004x over previous
import functools

import jax
import jax.numpy as jnp
from jax.experimental import pallas as pl
from jax.experimental.pallas import tpu as pltpu

BATCH = 4
SEQ_LEN = 8192
EMBED_DIM = 1024

SEQ_TILE = 2048
SEQ_TILES = SEQ_LEN // SEQ_TILE


def _add_body(x_ref, t_ref, o_ref):
    o_ref[...] = x_ref[...] + t_ref[...][None]


@jax.jit
def _tc_add(x, pos_table):
    return pl.pallas_call(
        _add_body,
        grid=(SEQ_TILES, BATCH),
        in_specs=[
            pl.BlockSpec((1, SEQ_TILE, EMBED_DIM), lambda s, b: (b, s, 0)),
            pl.BlockSpec((SEQ_TILE, EMBED_DIM), lambda s, b: (s, 0)),
        ],
        out_specs=pl.BlockSpec((1, SEQ_TILE, EMBED_DIM), lambda s, b: (b, s, 0)),
        out_shape=jax.ShapeDtypeStruct((BATCH, SEQ_LEN, EMBED_DIM), jnp.float32),
        compiler_params=pltpu.CompilerParams(
            dimension_semantics=("parallel", "arbitrary"),
        ),
    )(x, pos_table)


def kernel(x, pos_table):
    return _tc_add(x, pos_table)


# speedup vs baseline: 1.0034x; 1.0002x over previous
import functools

import jax
import jax.numpy as jnp
from jax.experimental import pallas as pl
from jax.experimental.pallas import tpu as pltpu

BATCH = 4
SEQ_LEN = 8192
EMBED_DIM = 1024

SEQ_TILE = 2048
SEQ_TILES = SEQ_LEN // SEQ_TILE


def _add_body(x_ref, t_ref, o_ref):
    o_ref[...] = x_ref[...] + t_ref[...][None]


@jax.jit
def _tc_add(x, pos_table):
    return pl.pallas_call(
        _add_body,
        grid=(SEQ_TILES, BATCH),
        in_specs=[
            pl.BlockSpec((1, SEQ_TILE, EMBED_DIM), lambda s, b: (b, s, 0)),
            pl.BlockSpec((SEQ_TILE, EMBED_DIM), lambda s, b: (s, 0)),
        ],
        out_specs=pl.BlockSpec((1, SEQ_TILE, EMBED_DIM), lambda s, b: (b, s, 0)),
        out_shape=jax.ShapeDtypeStruct((BATCH, SEQ_LEN, EMBED_DIM), jnp.float32),
        compiler_params=pltpu.CompilerParams(
            dimension_semantics=("arbitrary", "arbitrary"),
        ),
    )(x, pos_table)


def kernel(x, pos_table):
    return _tc_add(x, pos_table)
